# baseline (speedup 1.0000x reference)
import functools

import jax
import jax.numpy as jnp
from jax import lax
from jax.experimental import pallas as pl
from jax.experimental.pallas import tpu as pltpu
from jax.experimental.pallas import tpu_sc as plsc

B, F_IN, F_TGT, C, R = 4096, 480, 512, 8, 4
D = C * R
L = 16
NW = 32
B_PER_W = B // NW
N_PAD = F_TGT - F_IN
N_CHUNK = F_TGT // 128


def _pack_body(ll_hbm, perm_hbm, out_hbm,
               perm_v, sperm_v, idx_v, zpos_v, rows_v, sem):
    wid = lax.axis_index("s") * 2 + lax.axis_index("c")
    base_b = wid * B_PER_W

    pltpu.sync_copy(perm_hbm, perm_v)

    lane = jnp.arange(L, dtype=jnp.int32)
    for t in range(F_TGT // L):
        v = perm_v[pl.ds(t * L, L)]
        sperm_v[t // 8, pl.ds((t % 8) * L, L)] = jnp.where(v < F_IN, v, 0)
        plsc.store_scatter(zpos_v, [v - F_IN], lane + t * L, mask=v >= F_IN)

    def bloop(i, carry):
        b = base_b + i
        off = b * F_IN
        for t in range(F_TGT // L):
            idx_v[t // 8, pl.ds((t % 8) * L, L)] = (
                sperm_v[t // 8, pl.ds((t % 8) * L, L)] + off)

        copies = [
            pltpu.async_copy(ll_hbm.at[idx_v.at[k]],
                             rows_v.at[pl.ds(k * 128, 128)], sem)
            for k in range(N_CHUNK)
        ]
        for cp in copies:
            cp.wait()

        zv = jnp.zeros((L,), jnp.float32)
        for g in range(N_PAD // L):
            pp = zpos_v[pl.ds(g * L, L)]
            for col in range(D):
                plsc.store_scatter(
                    rows_v, [pp, jnp.full((L,), col, jnp.int32)], zv)

        pltpu.sync_copy(rows_v, out_hbm.at[pl.ds(b * F_TGT, F_TGT)])
        return carry

    lax.fori_loop(0, B_PER_W, bloop, 0)


def kernel(ll, perm):
    ll2 = ll.reshape(B * F_IN, D)
    mesh = plsc.VectorSubcoreMesh(core_axis_name="c", subcore_axis_name="s")
    out = pl.kernel(
        _pack_body,
        mesh=mesh,
        compiler_params=pltpu.CompilerParams(
            use_tc_tiling_on_sc=False, needs_layout_passes=False),
        out_type=jax.ShapeDtypeStruct((B * F_TGT, D), jnp.float32),
        scratch_types=[
            pltpu.VMEM((F_TGT,), jnp.int32),
            pltpu.VMEM((N_CHUNK, 128), jnp.int32),
            pltpu.VMEM((N_CHUNK, 128), jnp.int32),
            pltpu.VMEM((N_PAD,), jnp.int32),
            pltpu.VMEM((F_TGT, D), jnp.float32),
            pltpu.SemaphoreType.DMA,
        ],
    )(ll2, perm)
    return out.reshape(B, F_TGT, C, R)
